# Initial kernel scaffold; baseline (speedup 1.0000x reference)
import jax
import jax.numpy as jnp
from jax.experimental import pallas as pl


def kernel(input_ids, attention_mask, token_type_ids, pixel_values, pixel_mask, word_emb, pos_text_emb, tt_text_emb, ln_gamma, ln_beta, cls_token, vis_pos_emb, modality_emb, conv_w, conv_b):
    raise NotImplementedError("write your pallas kernel here")


# speedup vs baseline: 1.7224x; 1.7224x over previous
import functools

import jax
import jax.numpy as jnp
from jax import lax
from jax.experimental import pallas as pl
from jax.experimental.pallas import tpu as pltpu
from jax.experimental.pallas import tpu_sc as plsc

B = 32
L = 40
H = 768
V = 30522
IMG = 384
P = 32
PD = IMG // P
NP = PD * PD
K = 3 * P * P
SEQ = L + NP + 1

_NW = 32
_BPW = (B * L) // _NW


@functools.lru_cache(maxsize=1)
def _make_sc_gather():
    @functools.partial(
        pl.kernel,
        out_type=jax.ShapeDtypeStruct((B * L, H), jnp.float32),
        mesh=plsc.VectorSubcoreMesh(core_axis_name="c", subcore_axis_name="s"),
        scratch_types=[
            pltpu.VMEM((_BPW,), jnp.int32),
            pltpu.VMEM((_BPW, H), jnp.float32),
            pltpu.SemaphoreType.DMA,
        ],
    )
    def _sc_gather(table_hbm, idx_hbm, out_hbm, idx_v, rows_v, sem):
        wid = lax.axis_index("s") * 2 + lax.axis_index("c")
        base = wid * _BPW
        pltpu.sync_copy(idx_hbm.at[pl.ds(base, _BPW)], idx_v)
        pltpu.async_copy(table_hbm.at[idx_v], rows_v, sem).wait()
        pltpu.sync_copy(rows_v, out_hbm.at[pl.ds(base, _BPW)])

    return _sc_gather


def _tc_body(text_ref, ttf_ref, pos_ref, tt_ref, g_ref, b_ref, cls_ref,
             vpos_ref, mod_ref, patches_ref, w_ref, cb_ref, out_ref):
    x = text_ref[0] + pos_ref[...]
    t = ttf_ref[0]
    tt0 = tt_ref[0:1, :]
    tt1 = tt_ref[1:2, :]
    x = x + tt0 + t * (tt1 - tt0)
    m = jnp.mean(x, axis=-1, keepdims=True)
    xc = x - m
    v = jnp.mean(xc * xc, axis=-1, keepdims=True)
    x = xc * lax.rsqrt(v + 1e-12) * g_ref[...] + b_ref[...]
    mod0 = mod_ref[0:1, :]
    mod1 = mod_ref[1:2, :]
    out_ref[0, 0:L] = x + mod0
    acc = jnp.dot(patches_ref[0], w_ref[...], preferred_element_type=jnp.float32)
    vis = acc + cb_ref[...] + vpos_ref[1:NP + 1, :] + mod1
    out_ref[0, L:L + 1] = cls_ref[...] + vpos_ref[0:1, :] + mod1
    out_ref[0, L + 1:SEQ] = vis


def kernel(input_ids, attention_mask, token_type_ids, pixel_values, pixel_mask,
           word_emb, pos_text_emb, tt_text_emb, ln_gamma, ln_beta, cls_token,
           vis_pos_emb, modality_emb, conv_w, conv_b):
    patches = pixel_values.reshape(B, 3, PD, P, PD, P)
    patches = patches.transpose(0, 2, 4, 1, 3, 5).reshape(B, NP, K)
    wmat = conv_w.reshape(H, K).T
    ttf = token_type_ids.astype(jnp.float32).reshape(B, L, 1)

    gathered = _make_sc_gather()(word_emb, input_ids.reshape(-1)).reshape(B, L, H)

    emb = pl.pallas_call(
        _tc_body,
        grid=(B,),
        in_specs=[
            pl.BlockSpec((1, L, H), lambda b: (b, 0, 0)),
            pl.BlockSpec((1, L, 1), lambda b: (b, 0, 0)),
            pl.BlockSpec((L, H), lambda b: (0, 0)),
            pl.BlockSpec((2, H), lambda b: (0, 0)),
            pl.BlockSpec((1, H), lambda b: (0, 0)),
            pl.BlockSpec((1, H), lambda b: (0, 0)),
            pl.BlockSpec((1, H), lambda b: (0, 0)),
            pl.BlockSpec((NP + 1, H), lambda b: (0, 0)),
            pl.BlockSpec((2, H), lambda b: (0, 0)),
            pl.BlockSpec((1, NP, K), lambda b: (b, 0, 0)),
            pl.BlockSpec((K, H), lambda b: (0, 0)),
            pl.BlockSpec((1, H), lambda b: (0, 0)),
        ],
        out_specs=pl.BlockSpec((1, SEQ, H), lambda b: (b, 0, 0)),
        out_shape=jax.ShapeDtypeStruct((B, SEQ, H), jnp.float32),
        compiler_params=pltpu.CompilerParams(
            dimension_semantics=("arbitrary",),
        ),
    )(gathered, ttf, pos_text_emb, tt_text_emb, ln_gamma.reshape(1, H),
      ln_beta.reshape(1, H), cls_token.reshape(1, H),
      vis_pos_emb.reshape(NP + 1, H), modality_emb, patches, wmat,
      conv_b.reshape(1, H))

    masks = jnp.concatenate(
        [attention_mask,
         jnp.ones((B, NP + 1), dtype=attention_mask.dtype)], axis=1)
    return (emb, masks)


# speedup vs baseline: 1.7402x; 1.0104x over previous
import functools

import jax
import jax.numpy as jnp
from jax import lax
from jax.experimental import pallas as pl
from jax.experimental.pallas import tpu as pltpu
from jax.experimental.pallas import tpu_sc as plsc

B = 32
L = 40
H = 768
V = 30522
IMG = 384
P = 32
PD = IMG // P
NP = PD * PD
K = 3 * P * P
SEQ = L + NP + 1

_NW = 32
_BPW = (B * L) // _NW


@functools.lru_cache(maxsize=1)
def _make_sc_gather():
    @functools.partial(
        pl.kernel,
        out_type=jax.ShapeDtypeStruct((B * L, H), jnp.float32),
        mesh=plsc.VectorSubcoreMesh(core_axis_name="c", subcore_axis_name="s"),
        scratch_types=[
            pltpu.VMEM((_BPW,), jnp.int32),
            pltpu.VMEM((_BPW, H), jnp.float32),
            pltpu.SemaphoreType.DMA,
        ],
    )
    def _sc_gather(table_hbm, idx_hbm, out_hbm, idx_v, rows_v, sem):
        wid = lax.axis_index("s") * 2 + lax.axis_index("c")
        base = wid * _BPW
        pltpu.sync_copy(idx_hbm.at[pl.ds(base, _BPW)], idx_v)
        pltpu.async_copy(table_hbm.at[idx_v], rows_v, sem).wait()
        pltpu.sync_copy(rows_v, out_hbm.at[pl.ds(base, _BPW)])

    return _sc_gather


def _tc_body(text_ref, ttf_ref, pos_ref, tt_ref, g_ref, b_ref, cls_ref,
             vpos_ref, mod_ref, patches_ref, w_ref, cb_ref, out_ref):
    x = text_ref[0] + pos_ref[...]
    t = ttf_ref[0]
    tt0 = tt_ref[0:1, :]
    tt1 = tt_ref[1:2, :]
    x = x + tt0 + t * (tt1 - tt0)
    m = jnp.mean(x, axis=-1, keepdims=True)
    xc = x - m
    v = jnp.mean(xc * xc, axis=-1, keepdims=True)
    x = xc * lax.rsqrt(v + 1e-12) * g_ref[...] + b_ref[...]
    mod0 = mod_ref[0:1, :]
    mod1 = mod_ref[1:2, :]
    out_ref[0, 0:L] = x + mod0
    acc = jnp.dot(patches_ref[0], w_ref[...], preferred_element_type=jnp.float32)
    vis = acc + cb_ref[...] + vpos_ref[1:NP + 1, :] + mod1
    out_ref[0, L:L + 1] = cls_ref[...] + vpos_ref[0:1, :] + mod1
    out_ref[0, L + 1:SEQ] = vis


def kernel(input_ids, attention_mask, token_type_ids, pixel_values, pixel_mask,
           word_emb, pos_text_emb, tt_text_emb, ln_gamma, ln_beta, cls_token,
           vis_pos_emb, modality_emb, conv_w, conv_b):
    patches = pixel_values.astype(jnp.bfloat16).reshape(B, 3, PD, P, PD, P)
    patches = patches.transpose(0, 2, 4, 1, 3, 5).reshape(B, NP, K)
    wmat = conv_w.reshape(H, K).T.astype(jnp.bfloat16)
    ttf = token_type_ids.astype(jnp.float32).reshape(B, L, 1)

    gathered = _make_sc_gather()(word_emb, input_ids.reshape(-1)).reshape(B, L, H)

    emb = pl.pallas_call(
        _tc_body,
        grid=(B,),
        in_specs=[
            pl.BlockSpec((1, L, H), lambda b: (b, 0, 0)),
            pl.BlockSpec((1, L, 1), lambda b: (b, 0, 0)),
            pl.BlockSpec((L, H), lambda b: (0, 0)),
            pl.BlockSpec((2, H), lambda b: (0, 0)),
            pl.BlockSpec((1, H), lambda b: (0, 0)),
            pl.BlockSpec((1, H), lambda b: (0, 0)),
            pl.BlockSpec((1, H), lambda b: (0, 0)),
            pl.BlockSpec((NP + 1, H), lambda b: (0, 0)),
            pl.BlockSpec((2, H), lambda b: (0, 0)),
            pl.BlockSpec((1, NP, K), lambda b: (b, 0, 0)),
            pl.BlockSpec((K, H), lambda b: (0, 0)),
            pl.BlockSpec((1, H), lambda b: (0, 0)),
        ],
        out_specs=pl.BlockSpec((1, SEQ, H), lambda b: (b, 0, 0)),
        out_shape=jax.ShapeDtypeStruct((B, SEQ, H), jnp.float32),
        compiler_params=pltpu.CompilerParams(
            dimension_semantics=("arbitrary",),
        ),
    )(gathered, ttf, pos_text_emb, tt_text_emb, ln_gamma.reshape(1, H),
      ln_beta.reshape(1, H), cls_token.reshape(1, H),
      vis_pos_emb.reshape(NP + 1, H), modality_emb, patches, wmat,
      conv_b.reshape(1, H))

    masks = jnp.concatenate(
        [attention_mask,
         jnp.ones((B, NP + 1), dtype=attention_mask.dtype)], axis=1)
    return (emb, masks)


# speedup vs baseline: 5.1608x; 2.9656x over previous
import functools

import jax
import jax.numpy as jnp
from jax import lax
from jax.experimental import pallas as pl
from jax.experimental.pallas import tpu as pltpu
from jax.experimental.pallas import tpu_sc as plsc

B = 32
L = 40
H = 768
V = 30522
IMG = 384
P = 32
PD = IMG // P
NP = PD * PD
K = 3 * P * P
SEQ = L + NP + 1

_NW = 32
_BPW = (B * L) // _NW


@functools.lru_cache(maxsize=1)
def _make_sc_gather():
    @functools.partial(
        pl.kernel,
        out_type=jax.ShapeDtypeStruct((B * L, H), jnp.float32),
        mesh=plsc.VectorSubcoreMesh(core_axis_name="c", subcore_axis_name="s"),
        scratch_types=[
            pltpu.VMEM((_BPW,), jnp.int32),
            pltpu.VMEM((_BPW, H), jnp.float32),
            pltpu.SemaphoreType.DMA,
        ],
    )
    def _sc_gather(table_hbm, idx_hbm, out_hbm, idx_v, rows_v, sem):
        wid = lax.axis_index("s") * 2 + lax.axis_index("c")
        base = wid * _BPW
        pltpu.sync_copy(idx_hbm.at[pl.ds(base, _BPW)], idx_v)
        pltpu.async_copy(table_hbm.at[idx_v], rows_v, sem).wait()
        pltpu.sync_copy(rows_v, out_hbm.at[pl.ds(base, _BPW)])

    return _sc_gather


def _tc_body_px(text_ref, ttf_ref, pos_ref, tt_ref, g_ref, b_ref, cls_ref,
                vpos_ref, mod_ref, px_ref, w_ref, cb_ref, out_ref):
    x = text_ref[0] + pos_ref[...]
    t = ttf_ref[0]
    tt0 = tt_ref[0:1, :]
    tt1 = tt_ref[1:2, :]
    x = x + tt0 + t * (tt1 - tt0)
    m = jnp.mean(x, axis=-1, keepdims=True)
    xc = x - m
    v = jnp.mean(xc * xc, axis=-1, keepdims=True)
    x = xc * lax.rsqrt(v + 1e-12) * g_ref[...] + b_ref[...]
    mod0 = mod_ref[0:1, :]
    mod1 = mod_ref[1:2, :]
    out_ref[0, 0:L] = x + mod0
    px = px_ref[0]
    a = px.astype(jnp.bfloat16).reshape(3, PD, P, PD, P)
    a = a.transpose(1, 3, 0, 2, 4).reshape(NP, K)
    acc = jnp.dot(a, w_ref[...], preferred_element_type=jnp.float32)
    vis = acc + cb_ref[...] + vpos_ref[1:NP + 1, :] + mod1
    out_ref[0, L:L + 1] = cls_ref[...] + vpos_ref[0:1, :] + mod1
    out_ref[0, L + 1:SEQ] = vis


def _tc_body(text_ref, ttf_ref, pos_ref, tt_ref, g_ref, b_ref, cls_ref,
             vpos_ref, mod_ref, patches_ref, w_ref, cb_ref, out_ref):
    x = text_ref[0] + pos_ref[...]
    t = ttf_ref[0]
    tt0 = tt_ref[0:1, :]
    tt1 = tt_ref[1:2, :]
    x = x + tt0 + t * (tt1 - tt0)
    m = jnp.mean(x, axis=-1, keepdims=True)
    xc = x - m
    v = jnp.mean(xc * xc, axis=-1, keepdims=True)
    x = xc * lax.rsqrt(v + 1e-12) * g_ref[...] + b_ref[...]
    mod0 = mod_ref[0:1, :]
    mod1 = mod_ref[1:2, :]
    out_ref[0, 0:L] = x + mod0
    acc = jnp.dot(patches_ref[0], w_ref[...], preferred_element_type=jnp.float32)
    vis = acc + cb_ref[...] + vpos_ref[1:NP + 1, :] + mod1
    out_ref[0, L:L + 1] = cls_ref[...] + vpos_ref[0:1, :] + mod1
    out_ref[0, L + 1:SEQ] = vis


def kernel(input_ids, attention_mask, token_type_ids, pixel_values, pixel_mask,
           word_emb, pos_text_emb, tt_text_emb, ln_gamma, ln_beta, cls_token,
           vis_pos_emb, modality_emb, conv_w, conv_b):
    wmat = conv_w.reshape(H, K).T.astype(jnp.bfloat16)
    ttf = token_type_ids.astype(jnp.float32).reshape(B, L, 1)

    gathered = _make_sc_gather()(word_emb, input_ids.reshape(-1)).reshape(B, L, H)

    emb = pl.pallas_call(
        _tc_body_px,
        grid=(B,),
        in_specs=[
            pl.BlockSpec((1, L, H), lambda b: (b, 0, 0)),
            pl.BlockSpec((1, L, 1), lambda b: (b, 0, 0)),
            pl.BlockSpec((L, H), lambda b: (0, 0)),
            pl.BlockSpec((2, H), lambda b: (0, 0)),
            pl.BlockSpec((1, H), lambda b: (0, 0)),
            pl.BlockSpec((1, H), lambda b: (0, 0)),
            pl.BlockSpec((1, H), lambda b: (0, 0)),
            pl.BlockSpec((NP + 1, H), lambda b: (0, 0)),
            pl.BlockSpec((2, H), lambda b: (0, 0)),
            pl.BlockSpec((1, 3, IMG, IMG), lambda b: (b, 0, 0, 0)),
            pl.BlockSpec((K, H), lambda b: (0, 0)),
            pl.BlockSpec((1, H), lambda b: (0, 0)),
        ],
        out_specs=pl.BlockSpec((1, SEQ, H), lambda b: (b, 0, 0)),
        out_shape=jax.ShapeDtypeStruct((B, SEQ, H), jnp.float32),
        compiler_params=pltpu.CompilerParams(
            dimension_semantics=("arbitrary",),
        ),
    )(gathered, ttf, pos_text_emb, tt_text_emb, ln_gamma.reshape(1, H),
      ln_beta.reshape(1, H), cls_token.reshape(1, H),
      vis_pos_emb.reshape(NP + 1, H), modality_emb, pixel_values, wmat,
      conv_b.reshape(1, H))

    masks = jnp.concatenate(
        [attention_mask,
         jnp.ones((B, NP + 1), dtype=attention_mask.dtype)], axis=1)
    return (emb, masks)
